# Initial kernel scaffold; baseline (speedup 1.0000x reference)
import jax
import jax.numpy as jnp
from jax.experimental import pallas as pl


def kernel(con_x, cat_2, cat_3, cat_4, emb2_0, emb2_1, emb2_2, emb3_0, emb3_1, emb3_2, emb4, W1, b1, W2, b2, W3, b3):
    raise NotImplementedError("write your pallas kernel here")


# speedup vs baseline: 11.1761x; 11.1761x over previous
import jax
import jax.numpy as jnp
import numpy as np
from jax.experimental import pallas as pl
from jax.experimental.pallas import tpu as pltpu

_B = 16384
_BS = 2048

_PATTERN = np.array([0, 1, 0, 1, 0, 1,
                     0, 1, 2, 0, 1, 2, 0, 1, 2,
                     0, 1, 2, 3], dtype=np.int32)


def _fused_body(idx_ref, pat_ref, con_ref,
                e20_ref, e21_ref, e22_ref, e30_ref, e31_ref, e32_ref, e4_ref,
                W1_ref, b1_ref, W2_ref, b2_ref, W3_ref, b3_ref, out_ref):
    W1 = W1_ref[...]
    tstack = jnp.concatenate([
        jnp.dot(e20_ref[...], W1[0:4], preferred_element_type=jnp.float32),
        jnp.dot(e21_ref[...], W1[4:8], preferred_element_type=jnp.float32),
        jnp.dot(e22_ref[...], W1[8:12], preferred_element_type=jnp.float32),
        jnp.dot(e30_ref[...], W1[12:18], preferred_element_type=jnp.float32),
        jnp.dot(e31_ref[...], W1[18:24], preferred_element_type=jnp.float32),
        jnp.dot(e32_ref[...], W1[24:30], preferred_element_type=jnp.float32),
        jnp.dot(e4_ref[...], W1[30:38], preferred_element_type=jnp.float32),
    ], axis=0)

    onehot = (idx_ref[...] == pat_ref[...]).astype(jnp.float32)

    h = jnp.dot(onehot, tstack, preferred_element_type=jnp.float32)
    h += jnp.dot(con_ref[...], W1[38:44], preferred_element_type=jnp.float32)
    h = jnp.tanh(h + b1_ref[...])
    h = jnp.tanh(jnp.dot(h, W2_ref[...], preferred_element_type=jnp.float32)
                 + b2_ref[...])
    y = jnp.dot(h, W3_ref[...], preferred_element_type=jnp.float32) + b3_ref[...]
    out_ref[...] = jax.nn.sigmoid(y)


def kernel(con_x, cat_2, cat_3, cat_4,
           emb2_0, emb2_1, emb2_2, emb3_0, emb3_1, emb3_2, emb4,
           W1, b1, W2, b2, W3, b3):
    c2 = cat_2.astype(jnp.int32)
    c3 = cat_3.astype(jnp.int32)
    c4 = cat_4.astype(jnp.int32)
    idx = jnp.concatenate([
        jnp.repeat(c2, 2, axis=1),
        jnp.repeat(c3, 3, axis=1),
        jnp.repeat(c4, 4, axis=1),
    ], axis=1)

    b1r = b1.reshape(1, -1)
    b2r = b2.reshape(1, -1)
    b3r = b3.reshape(1, -1)

    B = con_x.shape[0]
    grid = (B // _BS,)

    def full(shape):
        nd = len(shape)
        return pl.BlockSpec(shape, lambda i: (0,) * nd)

    out = pl.pallas_call(
        _fused_body,
        grid=grid,
        in_specs=[
            pl.BlockSpec((_BS, 19), lambda i: (i, 0)),
            pl.BlockSpec((1, 19), lambda i: (0, 0)),
            pl.BlockSpec((_BS, 6), lambda i: (i, 0)),
            full(emb2_0.shape), full(emb2_1.shape), full(emb2_2.shape),
            full(emb3_0.shape), full(emb3_1.shape), full(emb3_2.shape),
            full(emb4.shape),
            full(W1.shape), full(b1r.shape),
            full(W2.shape), full(b2r.shape),
            full(W3.shape), full(b3r.shape),
        ],
        out_specs=pl.BlockSpec((_BS, 2), lambda i: (i, 0)),
        out_shape=jax.ShapeDtypeStruct((B, 2), jnp.float32),
        compiler_params=pltpu.CompilerParams(
            dimension_semantics=("arbitrary",),
        ),
    )(idx, jnp.asarray(_PATTERN)[None, :], con_x, emb2_0, emb2_1, emb2_2, emb3_0, emb3_1, emb3_2, emb4,
      W1, b1r, W2, b2r, W3, b3r)
    return out


# speedup vs baseline: 14.3414x; 1.2832x over previous
import jax
import jax.numpy as jnp
import numpy as np
from jax.experimental import pallas as pl
from jax.experimental.pallas import tpu as pltpu

_BS = 2048

_PATTERN = np.array([0, 1, 0, 1, 0, 1,
                     0, 1, 2, 0, 1, 2, 0, 1, 2,
                     0, 1, 2, 3] + [-1] * 6, dtype=np.float32)[None, :]


def _fused_body(x_ref, pat_ref,
                e20_ref, e21_ref, e22_ref, e30_ref, e31_ref, e32_ref, e4_ref,
                W1_ref, b1_ref, W2_ref, b2_ref, W3_ref, b3_ref, out_ref):
    W1 = W1_ref[...]
    wfold = jnp.concatenate([
        jnp.dot(e20_ref[...], W1[0:4], preferred_element_type=jnp.float32),
        jnp.dot(e21_ref[...], W1[4:8], preferred_element_type=jnp.float32),
        jnp.dot(e22_ref[...], W1[8:12], preferred_element_type=jnp.float32),
        jnp.dot(e30_ref[...], W1[12:18], preferred_element_type=jnp.float32),
        jnp.dot(e31_ref[...], W1[18:24], preferred_element_type=jnp.float32),
        jnp.dot(e32_ref[...], W1[24:30], preferred_element_type=jnp.float32),
        jnp.dot(e4_ref[...], W1[30:38], preferred_element_type=jnp.float32),
        W1[38:44],
    ], axis=0)

    x = x_ref[...]
    lane = jax.lax.broadcasted_iota(jnp.int32, x.shape, 1)
    x = jnp.where(lane < 19, (x == pat_ref[...]).astype(jnp.float32), x)

    h = jnp.dot(x, wfold, preferred_element_type=jnp.float32)
    h = jnp.tanh(h + b1_ref[...])
    h = jnp.tanh(jnp.dot(h, W2_ref[...], preferred_element_type=jnp.float32)
                 + b2_ref[...])
    y = jnp.dot(h, W3_ref[...], preferred_element_type=jnp.float32) + b3_ref[...]
    out_ref[...] = jax.nn.sigmoid(y)


def kernel(con_x, cat_2, cat_3, cat_4,
           emb2_0, emb2_1, emb2_2, emb3_0, emb3_1, emb3_2, emb4,
           W1, b1, W2, b2, W3, b3):
    x_packed = jnp.concatenate([
        jnp.repeat(cat_2.astype(jnp.float32), 2, axis=1),
        jnp.repeat(cat_3.astype(jnp.float32), 3, axis=1),
        jnp.repeat(cat_4.astype(jnp.float32), 4, axis=1),
        con_x,
    ], axis=1)

    b1r = b1.reshape(1, -1)
    b2r = b2.reshape(1, -1)
    b3r = b3.reshape(1, -1)

    B = con_x.shape[0]
    grid = (B // _BS,)

    def full(shape):
        nd = len(shape)
        return pl.BlockSpec(shape, lambda i: (0,) * nd)

    out = pl.pallas_call(
        _fused_body,
        grid=grid,
        in_specs=[
            pl.BlockSpec((_BS, 25), lambda i: (i, 0)),
            pl.BlockSpec((1, 25), lambda i: (0, 0)),
            full(emb2_0.shape), full(emb2_1.shape), full(emb2_2.shape),
            full(emb3_0.shape), full(emb3_1.shape), full(emb3_2.shape),
            full(emb4.shape),
            full(W1.shape), full(b1r.shape),
            full(W2.shape), full(b2r.shape),
            full(W3.shape), full(b3r.shape),
        ],
        out_specs=pl.BlockSpec((_BS, 2), lambda i: (i, 0)),
        out_shape=jax.ShapeDtypeStruct((B, 2), jnp.float32),
        compiler_params=pltpu.CompilerParams(
            dimension_semantics=("arbitrary",),
        ),
    )(x_packed, jnp.asarray(_PATTERN),
      emb2_0, emb2_1, emb2_2, emb3_0, emb3_1, emb3_2, emb4,
      W1, b1r, W2, b2r, W3, b3r)
    return out


# speedup vs baseline: 15.4196x; 1.0752x over previous
import jax
import jax.numpy as jnp
import numpy as np
from jax.experimental import pallas as pl
from jax.experimental.pallas import tpu as pltpu

_BS = 4096

_PATTERN = np.array([0, 1, 0, 1, 0, 1,
                     0, 1, 2, 0, 1, 2, 0, 1, 2,
                     0, 1, 2, 3] + [-1] * 6, dtype=np.float32)[None, :]


def _fused_body(x_ref, pat_ref,
                e20_ref, e21_ref, e22_ref, e30_ref, e31_ref, e32_ref, e4_ref,
                W1_ref, b1_ref, W2_ref, b2_ref, W3_ref, b3_ref, out_ref):
    W1 = W1_ref[...]
    wfold = jnp.concatenate([
        jnp.dot(e20_ref[...], W1[0:4], preferred_element_type=jnp.float32),
        jnp.dot(e21_ref[...], W1[4:8], preferred_element_type=jnp.float32),
        jnp.dot(e22_ref[...], W1[8:12], preferred_element_type=jnp.float32),
        jnp.dot(e30_ref[...], W1[12:18], preferred_element_type=jnp.float32),
        jnp.dot(e31_ref[...], W1[18:24], preferred_element_type=jnp.float32),
        jnp.dot(e32_ref[...], W1[24:30], preferred_element_type=jnp.float32),
        jnp.dot(e4_ref[...], W1[30:38], preferred_element_type=jnp.float32),
        W1[38:44],
    ], axis=0)

    x = x_ref[...]
    lane = jax.lax.broadcasted_iota(jnp.int32, x.shape, 1)
    x = jnp.where(lane < 19, (x == pat_ref[...]).astype(jnp.float32), x)

    h = jnp.dot(x, wfold, preferred_element_type=jnp.float32)
    h = jnp.tanh(h + b1_ref[...])
    h = jnp.tanh(jnp.dot(h, W2_ref[...], preferred_element_type=jnp.float32)
                 + b2_ref[...])
    y = jnp.dot(h, W3_ref[...], preferred_element_type=jnp.float32) + b3_ref[...]
    out_ref[...] = jax.nn.sigmoid(y)


def kernel(con_x, cat_2, cat_3, cat_4,
           emb2_0, emb2_1, emb2_2, emb3_0, emb3_1, emb3_2, emb4,
           W1, b1, W2, b2, W3, b3):
    x_packed = jnp.concatenate([
        jnp.repeat(cat_2.astype(jnp.float32), 2, axis=1),
        jnp.repeat(cat_3.astype(jnp.float32), 3, axis=1),
        jnp.repeat(cat_4.astype(jnp.float32), 4, axis=1),
        con_x,
    ], axis=1)

    b1r = b1.reshape(1, -1)
    b2r = b2.reshape(1, -1)
    b3r = b3.reshape(1, -1)

    B = con_x.shape[0]
    grid = (B // _BS,)

    def full(shape):
        nd = len(shape)
        return pl.BlockSpec(shape, lambda i: (0,) * nd)

    out = pl.pallas_call(
        _fused_body,
        grid=grid,
        in_specs=[
            pl.BlockSpec((_BS, 25), lambda i: (i, 0)),
            pl.BlockSpec((1, 25), lambda i: (0, 0)),
            full(emb2_0.shape), full(emb2_1.shape), full(emb2_2.shape),
            full(emb3_0.shape), full(emb3_1.shape), full(emb3_2.shape),
            full(emb4.shape),
            full(W1.shape), full(b1r.shape),
            full(W2.shape), full(b2r.shape),
            full(W3.shape), full(b3r.shape),
        ],
        out_specs=pl.BlockSpec((_BS, 2), lambda i: (i, 0)),
        out_shape=jax.ShapeDtypeStruct((B, 2), jnp.float32),
        compiler_params=pltpu.CompilerParams(
            dimension_semantics=("arbitrary",),
        ),
    )(x_packed, jnp.asarray(_PATTERN),
      emb2_0, emb2_1, emb2_2, emb3_0, emb3_1, emb3_2, emb4,
      W1, b1r, W2, b2r, W3, b3r)
    return out


# speedup vs baseline: 15.4755x; 1.0036x over previous
import jax
import jax.numpy as jnp
import numpy as np
from jax.experimental import pallas as pl
from jax.experimental.pallas import tpu as pltpu

_BS = 8192

_PATTERN = np.array([0, 1, 0, 1, 0, 1,
                     0, 1, 2, 0, 1, 2, 0, 1, 2,
                     0, 1, 2, 3] + [-1] * 6, dtype=np.float32)[None, :]


def _fused_body(x_ref, pat_ref,
                e20_ref, e21_ref, e22_ref, e30_ref, e31_ref, e32_ref, e4_ref,
                W1_ref, b1_ref, W2_ref, b2_ref, W3_ref, b3_ref, out_ref):
    W1 = W1_ref[...]
    wfold = jnp.concatenate([
        jnp.dot(e20_ref[...], W1[0:4], preferred_element_type=jnp.float32),
        jnp.dot(e21_ref[...], W1[4:8], preferred_element_type=jnp.float32),
        jnp.dot(e22_ref[...], W1[8:12], preferred_element_type=jnp.float32),
        jnp.dot(e30_ref[...], W1[12:18], preferred_element_type=jnp.float32),
        jnp.dot(e31_ref[...], W1[18:24], preferred_element_type=jnp.float32),
        jnp.dot(e32_ref[...], W1[24:30], preferred_element_type=jnp.float32),
        jnp.dot(e4_ref[...], W1[30:38], preferred_element_type=jnp.float32),
        W1[38:44],
    ], axis=0)

    x = x_ref[...]
    lane = jax.lax.broadcasted_iota(jnp.int32, x.shape, 1)
    x = jnp.where(lane < 19, (x == pat_ref[...]).astype(jnp.float32), x)

    h = jnp.dot(x, wfold, preferred_element_type=jnp.float32)
    h = jnp.tanh(h + b1_ref[...])
    h = jnp.tanh(jnp.dot(h, W2_ref[...], preferred_element_type=jnp.float32)
                 + b2_ref[...])
    y = jnp.dot(h, W3_ref[...], preferred_element_type=jnp.float32) + b3_ref[...]
    out_ref[...] = jax.nn.sigmoid(y)


def kernel(con_x, cat_2, cat_3, cat_4,
           emb2_0, emb2_1, emb2_2, emb3_0, emb3_1, emb3_2, emb4,
           W1, b1, W2, b2, W3, b3):
    x_packed = jnp.concatenate([
        jnp.repeat(cat_2.astype(jnp.float32), 2, axis=1),
        jnp.repeat(cat_3.astype(jnp.float32), 3, axis=1),
        jnp.repeat(cat_4.astype(jnp.float32), 4, axis=1),
        con_x,
    ], axis=1)

    b1r = b1.reshape(1, -1)
    b2r = b2.reshape(1, -1)
    b3r = b3.reshape(1, -1)

    B = con_x.shape[0]
    grid = (B // _BS,)

    def full(shape):
        nd = len(shape)
        return pl.BlockSpec(shape, lambda i: (0,) * nd)

    out = pl.pallas_call(
        _fused_body,
        grid=grid,
        in_specs=[
            pl.BlockSpec((_BS, 25), lambda i: (i, 0)),
            pl.BlockSpec((1, 25), lambda i: (0, 0)),
            full(emb2_0.shape), full(emb2_1.shape), full(emb2_2.shape),
            full(emb3_0.shape), full(emb3_1.shape), full(emb3_2.shape),
            full(emb4.shape),
            full(W1.shape), full(b1r.shape),
            full(W2.shape), full(b2r.shape),
            full(W3.shape), full(b3r.shape),
        ],
        out_specs=pl.BlockSpec((_BS, 2), lambda i: (i, 0)),
        out_shape=jax.ShapeDtypeStruct((B, 2), jnp.float32),
        compiler_params=pltpu.CompilerParams(
            dimension_semantics=("arbitrary",),
        ),
    )(x_packed, jnp.asarray(_PATTERN),
      emb2_0, emb2_1, emb2_2, emb3_0, emb3_1, emb3_2, emb4,
      W1, b1r, W2, b2r, W3, b3r)
    return out
